# Initial kernel scaffold; baseline (speedup 1.0000x reference)
import jax
import jax.numpy as jnp
from jax.experimental import pallas as pl


def kernel(x, edge_index, batch, W1, b1, W2, b2, Wc, bc, gamma, beta, Ws, bs, Wq1, bq1, Wq2, bq2, Wh1, bh1, gh, bh, Wh2, bh2, Wh3, bh3):
    raise NotImplementedError("write your pallas kernel here")


# speedup vs baseline: 2.7583x; 2.7583x over previous
import functools

import jax
import jax.numpy as jnp
import numpy as np
from jax import lax
from jax.experimental import pallas as pl
from jax.experimental.pallas import tpu as pltpu
from jax.experimental.pallas import tpu_sc as plsc

_N = 50000
_E = 800000
_H = 64
_BN = 1.0 / np.sqrt(1.0 + 1e-5)
_G = _E // 128
_NW = 32
_TBLK = 2000
_TGRID = _N // _TBLK
_NP = 50048
_TROWS = _NP // 16
_TCH = 136
_NPC = 51200
_CROWS = _NPC // 16

_f32 = jnp.float32


def _elu_tc(v):
    return jnp.where(v > 0, v, jnp.exp(v) - 1.0)


def _edge_body(hA0, hA1, hB0, hB1, row2, col2, out, agg, r1, c1, av, bv, zb):
    c = lax.axis_index("c")
    s = lax.axis_index("s")
    w = s * 2 + c
    g_lo = (w * _G) // _NW
    g_hi = ((w + 1) * _G) // _NW

    def zb_zero(i, carry):
        zb[i, pl.ds(0, 16)] = jnp.zeros((16,), _f32)
        zb[i, pl.ds(16, 16)] = jnp.zeros((16,), _f32)
        return carry
    lax.fori_loop(0, _TCH, zb_zero, 0)

    for p in range(2):
        hA = hA0 if p == 0 else hA1
        hB = hB0 if p == 0 else hB1
        for j in range(_TROWS // _TCH):
            pltpu.sync_copy(zb, agg.at[pl.ds(s * _TROWS + j * _TCH, _TCH)])
        plsc.subcore_barrier()

        def chunk(g, carry):
            pltpu.sync_copy(row2.at[g], r1)
            pltpu.sync_copy(col2.at[g], c1)
            pltpu.sync_copy(hA.at[r1], av)
            pltpu.sync_copy(hB.at[c1], bv)

            def comp(i, carry2):
                for hh in range(2):
                    sl = pl.ds(hh * 16, 16)
                    v = av[i, sl] + bv[i, sl]
                    av[i, sl] = jnp.where(v > 0, v, jnp.exp(v) - 1.0)
                return carry2
            lax.fori_loop(0, 128, comp, 0)
            pltpu.sync_copy(av, agg.at[r1], add=True)
            return carry
        lax.fori_loop(g_lo, g_hi, chunk, 0)
        plsc.subcore_barrier()
        off = s * _TROWS
        pltpu.sync_copy(agg.at[pl.ds(off, _TROWS)],
                        out.at[p, c, pl.ds(off, _TROWS)])
        plsc.subcore_barrier()


def _counts_body(row2, out0, out1, cnt, r1, ones_v, zb1):
    c = lax.axis_index("c")
    s = lax.axis_index("s")
    w = s * 2 + c
    g_lo = (w * _G) // _NW
    g_hi = ((w + 1) * _G) // _NW

    for j in range(8):
        ones_v[pl.ds(j * 16, 16)] = jnp.full((16,), 1.0, _f32)

    def z(i, carry):
        zb1[pl.ds(i * 16, 16)] = jnp.zeros((16,), _f32)
        return carry
    lax.fori_loop(0, _CROWS // 16, z, 0)

    pltpu.sync_copy(zb1, cnt.at[pl.ds(s * _CROWS, _CROWS)])
    plsc.subcore_barrier()

    def chunk(g, carry):
        pltpu.sync_copy(row2.at[g], r1)
        pltpu.sync_copy(ones_v, cnt.at[r1], add=True)
        return carry
    lax.fori_loop(g_lo, g_hi, chunk, 0)
    plsc.subcore_barrier()

    @pl.when(c == 0)
    def _():
        pltpu.sync_copy(cnt.at[pl.ds(s * _CROWS, _CROWS)],
                        out0.at[pl.ds(s * _CROWS, _CROWS)])

    @pl.when(c == 1)
    def _():
        pltpu.sync_copy(cnt.at[pl.ds(s * _CROWS, _CROWS)],
                        out1.at[pl.ds(s * _CROWS, _CROWS)])


_sc_mesh = plsc.VectorSubcoreMesh(core_axis_name="c", subcore_axis_name="s")

_edge_call = pl.kernel(
    _edge_body,
    out_type=jax.ShapeDtypeStruct((2, 2, _NP, 32), _f32),
    mesh=_sc_mesh,
    compiler_params=pltpu.CompilerParams(use_tc_tiling_on_sc=False),
    scratch_types=[
        pltpu.VMEM_SHARED((_NP, 32), _f32),
        pltpu.VMEM((128,), jnp.int32),
        pltpu.VMEM((128,), jnp.int32),
        pltpu.VMEM((128, 32), _f32),
        pltpu.VMEM((128, 32), _f32),
        pltpu.VMEM((_TCH, 32), _f32),
    ],
)

_counts_call = pl.kernel(
    _counts_body,
    out_type=[jax.ShapeDtypeStruct((_NPC,), _f32),
              jax.ShapeDtypeStruct((_NPC,), _f32)],
    mesh=_sc_mesh,
    compiler_params=pltpu.CompilerParams(use_tc_tiling_on_sc=False),
    scratch_types=[
        pltpu.VMEM_SHARED((_NPC,), _f32),
        pltpu.VMEM((128,), jnp.int32),
        pltpu.VMEM((128,), _f32),
        pltpu.VMEM((_CROWS,), _f32),
    ],
)


def _split_hab(hab, a0_o, a1_o, b0_o, b1_o):
    a0_o[...] = hab[:, 0:32]
    a1_o[...] = hab[:, 32:64]
    b0_o[...] = hab[:, 64:96]
    b1_o[...] = hab[:, 96:128]


def _node0_body(x_ref, w1, b1r, w2, b2r, ab, cb,
                h_out, a0_o, a1_o, b0_o, b1_o):
    xb = x_ref[...]
    h1 = _elu_tc(jnp.dot(xb, w1[...], preferred_element_type=_f32) + b1r[...])
    h = _elu_tc(jnp.dot(h1, w2[...], preferred_element_type=_f32) + b2r[...])
    h_out[...] = h
    hab = jnp.dot(h, ab[...], preferred_element_type=_f32) + cb[...]
    _split_hab(hab, a0_o, a1_o, b0_o, b1_o)


def _combine(p_ref, cnt_ref, g_ref, be_ref, h_ref):
    P = p_ref[...]
    cz = cnt_ref[...]
    craw = cz[0] + cz[1]
    cc = jnp.maximum(craw, 1.0)
    psum = jnp.concatenate([P[0, 0] + P[0, 1], P[1, 0] + P[1, 1]], axis=-1)
    beta_eff = jnp.where(craw > 0, be_ref[...], 0.0)
    return g_ref[...] * (psum / cc) + beta_eff + h_ref[...]


def _mid_body(p_ref, cnt_ref, g_ref, be_ref, h_ref, ab, cb,
              h_out, a0_o, a1_o, b0_o, b1_o):
    h = _combine(p_ref, cnt_ref, g_ref, be_ref, h_ref)
    h_out[...] = h
    hab = jnp.dot(h, ab[...], preferred_element_type=_f32) + cb[...]
    _split_hab(hab, a0_o, a1_o, b0_o, b1_o)


def _head_body(p_ref, cnt_ref, g_ref, be_ref, h_ref,
               ws, bsr, wq1, bq1r, wq2, bq2r,
               wh1, bh1r, ghs, bhr, wh2, bh2r, wh3, bh3r,
               con_o, log_o):
    h = _combine(p_ref, cnt_ref, g_ref, be_ref, h_ref)
    feats = _elu_tc(jnp.dot(h, ws[...], preferred_element_type=_f32) + bsr[...])
    cq = _elu_tc(jnp.dot(feats, wq1[...], preferred_element_type=_f32) + bq1r[...])
    con_o[...] = jnp.dot(cq, wq2[...], preferred_element_type=_f32) + bq2r[...]
    sv = _elu_tc(jnp.dot(feats, wh1[...], preferred_element_type=_f32) + bh1r[...])
    sv = sv * ghs[...] + bhr[...]
    sv = _elu_tc(jnp.dot(sv, wh2[...], preferred_element_type=_f32) + bh2r[...])
    log_o[...] = jnp.dot(sv, wh3[...], preferred_element_type=_f32) + bh3r[...]


def _row_spec(nc):
    return pl.BlockSpec((_TBLK, nc), lambda i: (i, 0))


def _w_spec(shape):
    nd = len(shape)
    return pl.BlockSpec(shape, lambda i: (0,) * nd)


_P_SPEC = pl.BlockSpec((2, 2, _TBLK, 32), lambda i: (0, 0, i, 0))
_CNT_SPEC = pl.BlockSpec((2, _TBLK, 1), lambda i: (0, i, 0))

_node0_call = pl.pallas_call(
    _node0_body,
    grid=(_TGRID,),
    in_specs=[_row_spec(16), _w_spec((16, 64)), _w_spec((1, 64)),
              _w_spec((64, 64)), _w_spec((1, 64)),
              _w_spec((64, 128)), _w_spec((1, 128))],
    out_specs=[_row_spec(64)] + [_row_spec(32)] * 4,
    out_shape=[jax.ShapeDtypeStruct((_N, 64), _f32)]
    + [jax.ShapeDtypeStruct((_N, 32), _f32)] * 4,
)

_mid_call = pl.pallas_call(
    _mid_body,
    grid=(_TGRID,),
    in_specs=[_P_SPEC, _CNT_SPEC, _w_spec((1, 64)), _w_spec((1, 64)),
              _row_spec(64), _w_spec((64, 128)), _w_spec((1, 128))],
    out_specs=[_row_spec(64)] + [_row_spec(32)] * 4,
    out_shape=[jax.ShapeDtypeStruct((_N, 64), _f32)]
    + [jax.ShapeDtypeStruct((_N, 32), _f32)] * 4,
)

_head_call = pl.pallas_call(
    _head_body,
    grid=(_TGRID,),
    in_specs=[_P_SPEC, _CNT_SPEC, _w_spec((1, 64)), _w_spec((1, 64)),
              _row_spec(64),
              _w_spec((64, 64)), _w_spec((1, 64)),
              _w_spec((64, 32)), _w_spec((1, 32)),
              _w_spec((32, 8)), _w_spec((1, 8)),
              _w_spec((64, 64)), _w_spec((1, 64)),
              _w_spec((1, 64)), _w_spec((1, 64)),
              _w_spec((64, 32)), _w_spec((1, 32)),
              _w_spec((32, 1)), _w_spec((1, 1))],
    out_specs=[_row_spec(8), _row_spec(1)],
    out_shape=[jax.ShapeDtypeStruct((_N, 8), _f32),
               jax.ShapeDtypeStruct((_N, 1), _f32)],
)


def kernel(x, edge_index, batch, W1, b1, W2, b2, Wc, bc, gamma, beta,
           Ws, bs, Wq1, bq1, Wq2, bq2, Wh1, bh1, gh, bh, Wh2, bh2, Wh3, bh3):
    row2 = edge_index[0].astype(jnp.int32).reshape(_G, 128)
    col2 = edge_index[1].astype(jnp.int32).reshape(_G, 128)

    cnt0, cnt1 = _counts_call(row2)
    cnt3 = jnp.stack([cnt0, cnt1]).reshape(2, _NPC, 1)

    A = Wc[:, :_H, :] - Wc[:, _H:, :]
    B = Wc[:, _H:, :]
    AB = jnp.concatenate([A, B], axis=2)
    CB = jnp.concatenate([bc, jnp.zeros_like(bc)], axis=1).reshape(4, 1, 128)
    G = (_BN * gamma).reshape(4, 1, 64)
    BE = beta.reshape(4, 1, 64)

    h, a0, a1, b0v, b1v = _node0_call(
        x, W1, b1.reshape(1, 64), W2, b2.reshape(1, 64), AB[0], CB[0])

    for i in range(4):
        P = _edge_call(a0, a1, b0v, b1v, row2, col2)
        if i < 3:
            h, a0, a1, b0v, b1v = _mid_call(
                P, cnt3, G[i], BE[i], h, AB[i + 1], CB[i + 1])
        else:
            con, logit = _head_call(
                P, cnt3, G[i], BE[i], h,
                Ws, bs.reshape(1, 64),
                Wq1, bq1.reshape(1, 32), Wq2, bq2.reshape(1, 8),
                Wh1, bh1.reshape(1, 64),
                (_BN * gh).reshape(1, 64), bh.reshape(1, 64),
                Wh2, bh2.reshape(1, 32), Wh3, bh3.reshape(1, 1))
    return (con, logit, batch)


# speedup vs baseline: 5.6948x; 2.0646x over previous
import functools

import jax
import jax.numpy as jnp
import numpy as np
from jax import lax
from jax.experimental import pallas as pl
from jax.experimental.pallas import tpu as pltpu
from jax.experimental.pallas import tpu_sc as plsc

_N = 50000
_E = 800000
_H = 64
_BN = 1.0 / np.sqrt(1.0 + 1e-5)
_G = _E // 128
_NW = 32
_TBLK = 2000
_TGRID = _N // _TBLK
_NP = 50048
_TROWS = _NP // 16
_TCH = 136
_NPC = 51200
_CROWS = _NPC // 16

_f32 = jnp.float32


def _elu_tc(v):
    return jnp.where(v > 0, v, jnp.exp(v) - 1.0)


_K = 6
_BE = _K * 128
_PW = 16
_NPASS = 64 // _PW


def _edge_body(hA0, hA1, hA2, hA3, hB0, hB1, hB2, hB3, row2, col2, out,
               agg, zb, r2a, c2a, ava, bva, r2b, c2b, avb, bvb, sga, sgb):
    c = lax.axis_index("c")
    s = lax.axis_index("s")
    w = s * 2 + c
    g_lo = (w * _G) // _NW
    g_hi = ((w + 1) * _G) // _NW
    nb = (g_hi - g_lo) // _K
    sets = ((r2a, c2a, ava, bva, sga), (r2b, c2b, avb, bvb, sgb))

    def zb_zero(i, carry):
        zb[i, pl.ds(0, 16)] = jnp.zeros((16,), _f32)
        return carry
    lax.fori_loop(0, _TCH, zb_zero, 0)

    def elu_inplace(av, bv, n_edges):
        @plsc.parallel_loop(0, n_edges, unroll=8)
        def _(i):
            sl = pl.ds(0, 16)
            v = av[i, sl] + bv[i, sl]
            av[i, sl] = jnp.where(v > 0, v, jnp.exp(v) - 1.0)

    for p in range(_NPASS):
        hA = (hA0, hA1, hA2, hA3)[p]
        hB = (hB0, hB1, hB2, hB3)[p]

        def g_start(bi, st):
            r2, c2, av, bv, sem = st
            base = g_lo + bi * _K
            pltpu.sync_copy(row2.at[pl.ds(base, _K)], r2)
            pltpu.sync_copy(col2.at[pl.ds(base, _K)], c2)
            for j in range(_K):
                pltpu.async_copy(hA.at[r2.at[j]],
                                 av.at[pl.ds(j * 128, 128)], sem)
                pltpu.async_copy(hB.at[c2.at[j]],
                                 bv.at[pl.ds(j * 128, 128)], sem)

        def g_finish_compute(st):
            r2, c2, av, bv, sem = st
            for j in range(_K):
                pltpu.make_async_copy(hA.at[r2.at[j]],
                                      av.at[pl.ds(j * 128, 128)], sem).wait()
                pltpu.make_async_copy(hB.at[c2.at[j]],
                                      bv.at[pl.ds(j * 128, 128)], sem).wait()
            elu_inplace(av, bv, _BE)
            for j in range(_K):
                pltpu.sync_copy(av.at[pl.ds(j * 128, 128)],
                                agg.at[r2.at[j]], add=True)

        for j in range(_TROWS // _TCH):
            pltpu.async_copy(zb, agg.at[pl.ds(s * _TROWS + j * _TCH, _TCH)],
                             sga)
        for j in range(_TROWS // _TCH):
            pltpu.make_async_copy(zb,
                                  agg.at[pl.ds(s * _TROWS + j * _TCH, _TCH)],
                                  sga).wait()
        plsc.subcore_barrier()

        @pl.when(nb > 0)
        def _():
            g_start(0, sets[0])

        def pair(k, carry):
            bi0 = 2 * k

            @pl.when(bi0 + 1 < nb)
            def _():
                g_start(bi0 + 1, sets[1])
            g_finish_compute(sets[0])

            @pl.when(bi0 + 2 < nb)
            def _():
                g_start(bi0 + 2, sets[0])

            @pl.when(bi0 + 1 < nb)
            def _():
                g_finish_compute(sets[1])
            return carry
        lax.fori_loop(0, (nb + 1) // 2, pair, 0)

        def tail(g, carry):
            r2, c2, av, bv, sem = sets[0]
            pltpu.sync_copy(row2.at[pl.ds(g, 1)], r2.at[pl.ds(0, 1)])
            pltpu.sync_copy(col2.at[pl.ds(g, 1)], c2.at[pl.ds(0, 1)])
            pltpu.sync_copy(hA.at[r2.at[0]], av.at[pl.ds(0, 128)])
            pltpu.sync_copy(hB.at[c2.at[0]], bv.at[pl.ds(0, 128)])
            elu_inplace(av, bv, 128)
            pltpu.sync_copy(av.at[pl.ds(0, 128)], agg.at[r2.at[0]], add=True)
            return carry
        lax.fori_loop(g_lo + nb * _K, g_hi, tail, 0)
        plsc.subcore_barrier()
        off = s * _TROWS
        pltpu.sync_copy(agg.at[pl.ds(off, _TROWS)],
                        out.at[p, c, pl.ds(off, _TROWS)])
        plsc.subcore_barrier()


def _counts_body(row2, out0, out1, cnt, r16, ones_v, zb1):
    c = lax.axis_index("c")
    s = lax.axis_index("s")
    w = s * 2 + c
    g_lo = (w * _G) // _NW
    g_hi = ((w + 1) * _G) // _NW
    nb = (g_hi - g_lo) // 16

    for j in range(8):
        ones_v[pl.ds(j * 16, 16)] = jnp.full((16,), 1.0, _f32)

    def z(i, carry):
        zb1[pl.ds(i * 16, 16)] = jnp.zeros((16,), _f32)
        return carry
    lax.fori_loop(0, _CROWS // 16, z, 0)

    pltpu.sync_copy(zb1, cnt.at[pl.ds(s * _CROWS, _CROWS)])
    plsc.subcore_barrier()

    def block(b, carry):
        pltpu.sync_copy(row2.at[pl.ds(g_lo + b * 16, 16)], r16)
        for j in range(16):
            pltpu.sync_copy(ones_v, cnt.at[r16.at[j]], add=True)
        return carry
    lax.fori_loop(0, nb, block, 0)

    def tail(g, carry):
        pltpu.sync_copy(row2.at[pl.ds(g, 1)], r16.at[pl.ds(0, 1)])
        pltpu.sync_copy(ones_v, cnt.at[r16.at[0]], add=True)
        return carry
    lax.fori_loop(g_lo + nb * 16, g_hi, tail, 0)
    plsc.subcore_barrier()

    @pl.when(c == 0)
    def _():
        pltpu.sync_copy(cnt.at[pl.ds(s * _CROWS, _CROWS)],
                        out0.at[pl.ds(s * _CROWS, _CROWS)])

    @pl.when(c == 1)
    def _():
        pltpu.sync_copy(cnt.at[pl.ds(s * _CROWS, _CROWS)],
                        out1.at[pl.ds(s * _CROWS, _CROWS)])


_sc_mesh = plsc.VectorSubcoreMesh(core_axis_name="c", subcore_axis_name="s")

_edge_call = pl.kernel(
    _edge_body,
    out_type=jax.ShapeDtypeStruct((_NPASS, 2, _NP, _PW), _f32),
    mesh=_sc_mesh,
    compiler_params=pltpu.CompilerParams(use_tc_tiling_on_sc=False),
    scratch_types=[
        pltpu.VMEM_SHARED((_NP, _PW), _f32),
        pltpu.VMEM((_TCH, _PW), _f32),
        pltpu.VMEM((_K, 128), jnp.int32),
        pltpu.VMEM((_K, 128), jnp.int32),
        pltpu.VMEM((_BE, _PW), _f32),
        pltpu.VMEM((_BE, _PW), _f32),
        pltpu.VMEM((_K, 128), jnp.int32),
        pltpu.VMEM((_K, 128), jnp.int32),
        pltpu.VMEM((_BE, _PW), _f32),
        pltpu.VMEM((_BE, _PW), _f32),
        pltpu.SemaphoreType.DMA,
        pltpu.SemaphoreType.DMA,
    ],
)

_counts_call = pl.kernel(
    _counts_body,
    out_type=[jax.ShapeDtypeStruct((_NPC,), _f32),
              jax.ShapeDtypeStruct((_NPC,), _f32)],
    mesh=_sc_mesh,
    compiler_params=pltpu.CompilerParams(use_tc_tiling_on_sc=False),
    scratch_types=[
        pltpu.VMEM_SHARED((_NPC,), _f32),
        pltpu.VMEM((16, 128), jnp.int32),
        pltpu.VMEM((128,), _f32),
        pltpu.VMEM((_CROWS,), _f32),
    ],
)


def _split_hab(hab, outs):
    for q in range(8):
        outs[q][...] = hab[:, q * 16:(q + 1) * 16]


def _node0_body(x_ref, w1, b1r, w2, b2r, ab, cb, h_out, *outs):
    xb = x_ref[...]
    h1 = _elu_tc(jnp.dot(xb, w1[...], preferred_element_type=_f32, precision=lax.Precision.HIGHEST) + b1r[...])
    h = _elu_tc(jnp.dot(h1, w2[...], preferred_element_type=_f32, precision=lax.Precision.HIGHEST) + b2r[...])
    h_out[...] = h
    hab = jnp.dot(h, ab[...], preferred_element_type=_f32, precision=lax.Precision.HIGHEST) + cb[...]
    _split_hab(hab, outs)


def _combine(p_ref, cnt_ref, g_ref, be_ref, h_ref):
    P = p_ref[...]
    cz = cnt_ref[...]
    craw = cz[0] + cz[1]
    cc = jnp.maximum(craw, 1.0)
    psum = jnp.concatenate([P[q, 0] + P[q, 1] for q in range(_NPASS)],
                           axis=-1)
    beta_eff = jnp.where(craw > 0, be_ref[...], 0.0)
    return g_ref[...] * (psum / cc) + beta_eff + h_ref[...]


def _mid_body(p_ref, cnt_ref, g_ref, be_ref, h_ref, ab, cb, h_out, *outs):
    h = _combine(p_ref, cnt_ref, g_ref, be_ref, h_ref)
    h_out[...] = h
    hab = jnp.dot(h, ab[...], preferred_element_type=_f32, precision=lax.Precision.HIGHEST) + cb[...]
    _split_hab(hab, outs)


def _head_body(p_ref, cnt_ref, g_ref, be_ref, h_ref,
               ws, bsr, wq1, bq1r, wq2, bq2r,
               wh1, bh1r, ghs, bhr, wh2, bh2r, wh3, bh3r,
               con_o, log_o):
    h = _combine(p_ref, cnt_ref, g_ref, be_ref, h_ref)
    feats = _elu_tc(jnp.dot(h, ws[...], preferred_element_type=_f32, precision=lax.Precision.HIGHEST) + bsr[...])
    cq = _elu_tc(jnp.dot(feats, wq1[...], preferred_element_type=_f32, precision=lax.Precision.HIGHEST) + bq1r[...])
    con_o[...] = jnp.dot(cq, wq2[...], preferred_element_type=_f32, precision=lax.Precision.HIGHEST) + bq2r[...]
    sv = _elu_tc(jnp.dot(feats, wh1[...], preferred_element_type=_f32, precision=lax.Precision.HIGHEST) + bh1r[...])
    sv = sv * ghs[...] + bhr[...]
    sv = _elu_tc(jnp.dot(sv, wh2[...], preferred_element_type=_f32, precision=lax.Precision.HIGHEST) + bh2r[...])
    log_o[...] = jnp.dot(sv, wh3[...], preferred_element_type=_f32, precision=lax.Precision.HIGHEST) + bh3r[...]


def _row_spec(nc):
    return pl.BlockSpec((_TBLK, nc), lambda i: (i, 0))


def _w_spec(shape):
    nd = len(shape)
    return pl.BlockSpec(shape, lambda i: (0,) * nd)


_P_SPEC = pl.BlockSpec((_NPASS, 2, _TBLK, _PW), lambda i: (0, 0, i, 0))
_CNT_SPEC = pl.BlockSpec((2, _TBLK, 1), lambda i: (0, i, 0))

_node0_call = pl.pallas_call(
    _node0_body,
    grid=(_TGRID,),
    in_specs=[_row_spec(16), _w_spec((16, 64)), _w_spec((1, 64)),
              _w_spec((64, 64)), _w_spec((1, 64)),
              _w_spec((64, 128)), _w_spec((1, 128))],
    out_specs=[_row_spec(64)] + [_row_spec(16)] * 8,
    out_shape=[jax.ShapeDtypeStruct((_N, 64), _f32)]
    + [jax.ShapeDtypeStruct((_N, 16), _f32)] * 8,
)

_mid_call = pl.pallas_call(
    _mid_body,
    grid=(_TGRID,),
    in_specs=[_P_SPEC, _CNT_SPEC, _w_spec((1, 64)), _w_spec((1, 64)),
              _row_spec(64), _w_spec((64, 128)), _w_spec((1, 128))],
    out_specs=[_row_spec(64)] + [_row_spec(16)] * 8,
    out_shape=[jax.ShapeDtypeStruct((_N, 64), _f32)]
    + [jax.ShapeDtypeStruct((_N, 16), _f32)] * 8,
)

_head_call = pl.pallas_call(
    _head_body,
    grid=(_TGRID,),
    in_specs=[_P_SPEC, _CNT_SPEC, _w_spec((1, 64)), _w_spec((1, 64)),
              _row_spec(64),
              _w_spec((64, 64)), _w_spec((1, 64)),
              _w_spec((64, 32)), _w_spec((1, 32)),
              _w_spec((32, 8)), _w_spec((1, 8)),
              _w_spec((64, 64)), _w_spec((1, 64)),
              _w_spec((1, 64)), _w_spec((1, 64)),
              _w_spec((64, 32)), _w_spec((1, 32)),
              _w_spec((32, 1)), _w_spec((1, 1))],
    out_specs=[_row_spec(8), _row_spec(1)],
    out_shape=[jax.ShapeDtypeStruct((_N, 8), _f32),
               jax.ShapeDtypeStruct((_N, 1), _f32)],
)


def kernel(x, edge_index, batch, W1, b1, W2, b2, Wc, bc, gamma, beta,
           Ws, bs, Wq1, bq1, Wq2, bq2, Wh1, bh1, gh, bh, Wh2, bh2, Wh3, bh3):
    row2 = edge_index[0].astype(jnp.int32).reshape(_G, 128)
    col2 = edge_index[1].astype(jnp.int32).reshape(_G, 128)

    cnt0, cnt1 = _counts_call(row2)
    cnt3 = jnp.stack([cnt0, cnt1]).reshape(2, _NPC, 1)

    A = Wc[:, :_H, :] - Wc[:, _H:, :]
    B = Wc[:, _H:, :]
    AB = jnp.concatenate([A, B], axis=2)
    CB = jnp.concatenate([bc, jnp.zeros_like(bc)], axis=1).reshape(4, 1, 128)
    G = (_BN * gamma).reshape(4, 1, 64)
    BE = beta.reshape(4, 1, 64)

    h, *tabs = _node0_call(
        x, W1, b1.reshape(1, 64), W2, b2.reshape(1, 64), AB[0], CB[0])

    for i in range(4):
        P = _edge_call(*tabs, row2, col2)
        if i < 3:
            h, *tabs = _mid_call(
                P, cnt3, G[i], BE[i], h, AB[i + 1], CB[i + 1])
        else:
            con, logit = _head_call(
                P, cnt3, G[i], BE[i], h,
                Ws, bs.reshape(1, 64),
                Wq1, bq1.reshape(1, 32), Wq2, bq2.reshape(1, 8),
                Wh1, bh1.reshape(1, 64),
                (_BN * gh).reshape(1, 64), bh.reshape(1, 64),
                Wh2, bh2.reshape(1, 32), Wh3, bh3.reshape(1, 1))
    return (con, logit, batch)


# speedup vs baseline: 8.0853x; 1.4198x over previous
import functools

import jax
import jax.numpy as jnp
import numpy as np
from jax import lax
from jax.experimental import pallas as pl
from jax.experimental.pallas import tpu as pltpu
from jax.experimental.pallas import tpu_sc as plsc

_N = 50000
_E = 800000
_H = 64
_BN = 1.0 / np.sqrt(1.0 + 1e-5)
_G = _E // 128
_NW = 32
_TBLK = 2000
_TGRID = _N // _TBLK
_NP = 50048
_TROWS = _NP // 16
_TCH = 136
_NPC = 51200
_CROWS = _NPC // 16

_f32 = jnp.float32


def _elu_tc(v):
    return jnp.where(v > 0, v, jnp.exp(v) - 1.0)


_K = 6
_BE = _K * 128
_PW = 16
_NPASS = 64 // _PW


def _edge_body(tab, row2, col2, out, agg, zb,
               r2a, c2a, ga, ha, ava, bva,
               r2b, c2b, gb, hb, avb, bvb, sga, sgb):
    c = lax.axis_index("c")
    s = lax.axis_index("s")
    w = s * 2 + c
    g_lo = (w * _G) // _NW
    g_hi = ((w + 1) * _G) // _NW
    nb = (g_hi - g_lo) // _K
    sets = ((r2a, c2a, ga, ha, ava, bva, sga),
            (r2b, c2b, gb, hb, avb, bvb, sgb))

    def zb_zero(i, carry):
        zb[i, pl.ds(0, 16)] = jnp.zeros((16,), _f32)
        return carry
    lax.fori_loop(0, _TCH, zb_zero, 0)

    def elu_inplace(av, bv, n_edges):
        @plsc.parallel_loop(0, n_edges, unroll=8)
        def _(i):
            sl = pl.ds(0, 16)
            v = av[i, sl] + bv[i, sl]
            av[i, sl] = jnp.where(v > 0, v, jnp.exp(v) - 1.0)

    for p in range(_NPASS):
        qa = p
        qb = 4 + p

        def g_start(bi, st):
            r2, c2, gi, hi, av, bv, sem = st
            base = g_lo + bi * _K
            pltpu.sync_copy(row2.at[pl.ds(base, _K)], r2)
            pltpu.sync_copy(col2.at[pl.ds(base, _K)], c2)
            for j in range(_K):
                for u in range(8):
                    sl = pl.ds(u * 16, 16)
                    gi[j, sl] = r2[j, sl] * 8 + qa
                    hi[j, sl] = c2[j, sl] * 8 + qb
            for j in range(_K):
                pltpu.async_copy(tab.at[gi.at[j]],
                                 av.at[pl.ds(j * 128, 128)], sem)
                pltpu.async_copy(tab.at[hi.at[j]],
                                 bv.at[pl.ds(j * 128, 128)], sem)

        def g_finish_compute(st):
            r2, c2, gi, hi, av, bv, sem = st
            for j in range(_K):
                pltpu.make_async_copy(tab.at[gi.at[j]],
                                      av.at[pl.ds(j * 128, 128)], sem).wait()
                pltpu.make_async_copy(tab.at[hi.at[j]],
                                      bv.at[pl.ds(j * 128, 128)], sem).wait()
            elu_inplace(av, bv, _BE)
            for j in range(_K):
                pltpu.sync_copy(av.at[pl.ds(j * 128, 128)],
                                agg.at[r2.at[j]], add=True)

        for j in range(_TROWS // _TCH):
            pltpu.async_copy(zb, agg.at[pl.ds(s * _TROWS + j * _TCH, _TCH)],
                             sga)
        for j in range(_TROWS // _TCH):
            pltpu.make_async_copy(zb,
                                  agg.at[pl.ds(s * _TROWS + j * _TCH, _TCH)],
                                  sga).wait()
        plsc.subcore_barrier()

        @pl.when(nb > 0)
        def _():
            g_start(0, sets[0])

        def pair(k, carry):
            bi0 = 2 * k

            @pl.when(bi0 + 1 < nb)
            def _():
                g_start(bi0 + 1, sets[1])
            g_finish_compute(sets[0])

            @pl.when(bi0 + 2 < nb)
            def _():
                g_start(bi0 + 2, sets[0])

            @pl.when(bi0 + 1 < nb)
            def _():
                g_finish_compute(sets[1])
            return carry
        lax.fori_loop(0, (nb + 1) // 2, pair, 0)

        def tail(g, carry):
            r2, c2, gi, hi, av, bv, sem = sets[0]
            pltpu.sync_copy(row2.at[pl.ds(g, 1)], r2.at[pl.ds(0, 1)])
            pltpu.sync_copy(col2.at[pl.ds(g, 1)], c2.at[pl.ds(0, 1)])
            for u in range(8):
                sl = pl.ds(u * 16, 16)
                gi[0, sl] = r2[0, sl] * 8 + qa
                hi[0, sl] = c2[0, sl] * 8 + qb
            pltpu.sync_copy(tab.at[gi.at[0]], av.at[pl.ds(0, 128)])
            pltpu.sync_copy(tab.at[hi.at[0]], bv.at[pl.ds(0, 128)])
            elu_inplace(av, bv, 128)
            pltpu.sync_copy(av.at[pl.ds(0, 128)], agg.at[r2.at[0]], add=True)
            return carry
        lax.fori_loop(g_lo + nb * _K, g_hi, tail, 0)
        plsc.subcore_barrier()
        off = s * _TROWS
        pltpu.sync_copy(agg.at[pl.ds(off, _TROWS)],
                        out.at[pl.ds(off, _TROWS),
                               pl.ds(c * 64 + p * 16, 16)])
        plsc.subcore_barrier()


def _counts_body(row2, out0, out1, cnt, r16, ones_v, zb1):
    c = lax.axis_index("c")
    s = lax.axis_index("s")
    w = s * 2 + c
    g_lo = (w * _G) // _NW
    g_hi = ((w + 1) * _G) // _NW
    nb = (g_hi - g_lo) // 16

    for j in range(8):
        ones_v[pl.ds(j * 16, 16)] = jnp.full((16,), 1.0, _f32)

    def z(i, carry):
        zb1[pl.ds(i * 16, 16)] = jnp.zeros((16,), _f32)
        return carry
    lax.fori_loop(0, _CROWS // 16, z, 0)

    pltpu.sync_copy(zb1, cnt.at[pl.ds(s * _CROWS, _CROWS)])
    plsc.subcore_barrier()

    def block(b, carry):
        pltpu.sync_copy(row2.at[pl.ds(g_lo + b * 16, 16)], r16)
        for j in range(16):
            pltpu.sync_copy(ones_v, cnt.at[r16.at[j]], add=True)
        return carry
    lax.fori_loop(0, nb, block, 0)

    def tail(g, carry):
        pltpu.sync_copy(row2.at[pl.ds(g, 1)], r16.at[pl.ds(0, 1)])
        pltpu.sync_copy(ones_v, cnt.at[r16.at[0]], add=True)
        return carry
    lax.fori_loop(g_lo + nb * 16, g_hi, tail, 0)
    plsc.subcore_barrier()

    @pl.when(c == 0)
    def _():
        pltpu.sync_copy(cnt.at[pl.ds(s * _CROWS, _CROWS)],
                        out0.at[pl.ds(s * _CROWS, _CROWS)])

    @pl.when(c == 1)
    def _():
        pltpu.sync_copy(cnt.at[pl.ds(s * _CROWS, _CROWS)],
                        out1.at[pl.ds(s * _CROWS, _CROWS)])


_sc_mesh = plsc.VectorSubcoreMesh(core_axis_name="c", subcore_axis_name="s")

_edge_call = pl.kernel(
    _edge_body,
    out_type=jax.ShapeDtypeStruct((_NP, 128), _f32),
    mesh=_sc_mesh,
    compiler_params=pltpu.CompilerParams(use_tc_tiling_on_sc=False),
    scratch_types=[
        pltpu.VMEM_SHARED((_NP, _PW), _f32),
        pltpu.VMEM((_TCH, _PW), _f32),
        pltpu.VMEM((_K, 128), jnp.int32),
        pltpu.VMEM((_K, 128), jnp.int32),
        pltpu.VMEM((_K, 128), jnp.int32),
        pltpu.VMEM((_K, 128), jnp.int32),
        pltpu.VMEM((_BE, _PW), _f32),
        pltpu.VMEM((_BE, _PW), _f32),
        pltpu.VMEM((_K, 128), jnp.int32),
        pltpu.VMEM((_K, 128), jnp.int32),
        pltpu.VMEM((_K, 128), jnp.int32),
        pltpu.VMEM((_K, 128), jnp.int32),
        pltpu.VMEM((_BE, _PW), _f32),
        pltpu.VMEM((_BE, _PW), _f32),
        pltpu.SemaphoreType.DMA,
        pltpu.SemaphoreType.DMA,
    ],
)

_counts_call = pl.kernel(
    _counts_body,
    out_type=[jax.ShapeDtypeStruct((_NPC,), _f32),
              jax.ShapeDtypeStruct((_NPC,), _f32)],
    mesh=_sc_mesh,
    compiler_params=pltpu.CompilerParams(use_tc_tiling_on_sc=False),
    scratch_types=[
        pltpu.VMEM_SHARED((_NPC,), _f32),
        pltpu.VMEM((16, 128), jnp.int32),
        pltpu.VMEM((128,), _f32),
        pltpu.VMEM((_CROWS,), _f32),
    ],
)


def _node0_body(x_ref, w1, b1r, w2, b2r, ab, cb, h_out, hab_out):
    xb = x_ref[...]
    h1 = _elu_tc(jnp.dot(xb, w1[...], preferred_element_type=_f32, precision=lax.Precision.HIGHEST) + b1r[...])
    h = _elu_tc(jnp.dot(h1, w2[...], preferred_element_type=_f32, precision=lax.Precision.HIGHEST) + b2r[...])
    h_out[...] = h
    hab_out[...] = jnp.dot(h, ab[...], preferred_element_type=_f32, precision=lax.Precision.HIGHEST) + cb[...]


def _combine(p_ref, cnt_ref, g_ref, be_ref, h_ref):
    P = p_ref[...]
    cz = cnt_ref[...]
    craw = cz[0] + cz[1]
    cc = jnp.maximum(craw, 1.0)
    psum = P[:, 0:64] + P[:, 64:128]
    beta_eff = jnp.where(craw > 0, be_ref[...], 0.0)
    return g_ref[...] * (psum / cc) + beta_eff + h_ref[...]


def _mid_body(p_ref, cnt_ref, g_ref, be_ref, h_ref, ab, cb, h_out, hab_out):
    h = _combine(p_ref, cnt_ref, g_ref, be_ref, h_ref)
    h_out[...] = h
    hab_out[...] = jnp.dot(h, ab[...], preferred_element_type=_f32, precision=lax.Precision.HIGHEST) + cb[...]


def _head_body(p_ref, cnt_ref, g_ref, be_ref, h_ref,
               ws, bsr, wq1, bq1r, wq2, bq2r,
               wh1, bh1r, ghs, bhr, wh2, bh2r, wh3, bh3r,
               con_o, log_o):
    h = _combine(p_ref, cnt_ref, g_ref, be_ref, h_ref)
    feats = _elu_tc(jnp.dot(h, ws[...], preferred_element_type=_f32, precision=lax.Precision.HIGHEST) + bsr[...])
    cq = _elu_tc(jnp.dot(feats, wq1[...], preferred_element_type=_f32, precision=lax.Precision.HIGHEST) + bq1r[...])
    con_o[...] = jnp.dot(cq, wq2[...], preferred_element_type=_f32, precision=lax.Precision.HIGHEST) + bq2r[...]
    sv = _elu_tc(jnp.dot(feats, wh1[...], preferred_element_type=_f32, precision=lax.Precision.HIGHEST) + bh1r[...])
    sv = sv * ghs[...] + bhr[...]
    sv = _elu_tc(jnp.dot(sv, wh2[...], preferred_element_type=_f32, precision=lax.Precision.HIGHEST) + bh2r[...])
    log_o[...] = jnp.dot(sv, wh3[...], preferred_element_type=_f32, precision=lax.Precision.HIGHEST) + bh3r[...]


def _row_spec(nc):
    return pl.BlockSpec((_TBLK, nc), lambda i: (i, 0))


def _w_spec(shape):
    nd = len(shape)
    return pl.BlockSpec(shape, lambda i: (0,) * nd)


_P_SPEC = pl.BlockSpec((_TBLK, 128), lambda i: (i, 0))
_CNT_SPEC = pl.BlockSpec((2, _TBLK, 1), lambda i: (0, i, 0))

_node0_call = pl.pallas_call(
    _node0_body,
    grid=(_TGRID,),
    in_specs=[_row_spec(16), _w_spec((16, 64)), _w_spec((1, 64)),
              _w_spec((64, 64)), _w_spec((1, 64)),
              _w_spec((64, 128)), _w_spec((1, 128))],
    out_specs=[_row_spec(64), _row_spec(128)],
    out_shape=[jax.ShapeDtypeStruct((_N, 64), _f32),
               jax.ShapeDtypeStruct((_N, 128), _f32)],
)

_mid_call = pl.pallas_call(
    _mid_body,
    grid=(_TGRID,),
    in_specs=[_P_SPEC, _CNT_SPEC, _w_spec((1, 64)), _w_spec((1, 64)),
              _row_spec(64), _w_spec((64, 128)), _w_spec((1, 128))],
    out_specs=[_row_spec(64), _row_spec(128)],
    out_shape=[jax.ShapeDtypeStruct((_N, 64), _f32),
               jax.ShapeDtypeStruct((_N, 128), _f32)],
)

_head_call = pl.pallas_call(
    _head_body,
    grid=(_TGRID,),
    in_specs=[_P_SPEC, _CNT_SPEC, _w_spec((1, 64)), _w_spec((1, 64)),
              _row_spec(64),
              _w_spec((64, 64)), _w_spec((1, 64)),
              _w_spec((64, 32)), _w_spec((1, 32)),
              _w_spec((32, 8)), _w_spec((1, 8)),
              _w_spec((64, 64)), _w_spec((1, 64)),
              _w_spec((1, 64)), _w_spec((1, 64)),
              _w_spec((64, 32)), _w_spec((1, 32)),
              _w_spec((32, 1)), _w_spec((1, 1))],
    out_specs=[_row_spec(8), _row_spec(1)],
    out_shape=[jax.ShapeDtypeStruct((_N, 8), _f32),
               jax.ShapeDtypeStruct((_N, 1), _f32)],
)


def kernel(x, edge_index, batch, W1, b1, W2, b2, Wc, bc, gamma, beta,
           Ws, bs, Wq1, bq1, Wq2, bq2, Wh1, bh1, gh, bh, Wh2, bh2, Wh3, bh3):
    row2 = edge_index[0].astype(jnp.int32).reshape(_G, 128)
    col2 = edge_index[1].astype(jnp.int32).reshape(_G, 128)

    cnt0, cnt1 = _counts_call(row2)
    cnt3 = jnp.stack([cnt0, cnt1]).reshape(2, _NPC, 1)

    A = Wc[:, :_H, :] - Wc[:, _H:, :]
    B = Wc[:, _H:, :]
    AB = jnp.concatenate([A, B], axis=2)
    CB = jnp.concatenate([bc, jnp.zeros_like(bc)], axis=1).reshape(4, 1, 128)
    G = (_BN * gamma).reshape(4, 1, 64)
    BE = beta.reshape(4, 1, 64)

    h, hab = _node0_call(
        x, W1, b1.reshape(1, 64), W2, b2.reshape(1, 64), AB[0], CB[0])

    for i in range(4):
        P = _edge_call(hab.reshape(8 * _N, 16), row2, col2)
        if i < 3:
            h, hab = _mid_call(
                P, cnt3, G[i], BE[i], h, AB[i + 1], CB[i + 1])
        else:
            con, logit = _head_call(
                P, cnt3, G[i], BE[i], h,
                Ws, bs.reshape(1, 64),
                Wq1, bq1.reshape(1, 32), Wq2, bq2.reshape(1, 8),
                Wh1, bh1.reshape(1, 64),
                (_BN * gh).reshape(1, 64), bh.reshape(1, 64),
                Wh2, bh2.reshape(1, 32), Wh3, bh3.reshape(1, 1))
    return (con, logit, batch)


# speedup vs baseline: 8.5171x; 1.0534x over previous
import functools

import jax
import jax.numpy as jnp
import numpy as np
from jax import lax
from jax.experimental import pallas as pl
from jax.experimental.pallas import tpu as pltpu
from jax.experimental.pallas import tpu_sc as plsc

_N = 50000
_E = 800000
_H = 64
_BN = 1.0 / np.sqrt(1.0 + 1e-5)
_G = _E // 128
_NW = 32
_TBLK = 2000
_TGRID = _N // _TBLK
_NP = 50048
_TROWS = _NP // 16
_TCH = 136
_NPC = 51200
_CROWS = _NPC // 16

_f32 = jnp.float32


def _elu_tc(v):
    return jnp.where(v > 0, v, jnp.exp(v) - 1.0)


_K = 8
_BE = _K * 128
_PW = 16
_NPASS = 64 // _PW


def _edge_body(tab, row2, col2, out, agg, zb,
               r2a, c2a, ga, ha, ava, bva,
               r2b, c2b, gb, hb, avb, bvb, sga, sgb):
    c = lax.axis_index("c")
    s = lax.axis_index("s")
    w = s * 2 + c
    g_lo = (w * _G) // _NW
    g_hi = ((w + 1) * _G) // _NW
    nb = (g_hi - g_lo) // _K
    sets = ((r2a, c2a, ga, ha, ava, bva, sga),
            (r2b, c2b, gb, hb, avb, bvb, sgb))

    def zb_zero(i, carry):
        zb[i, pl.ds(0, 16)] = jnp.zeros((16,), _f32)
        return carry
    lax.fori_loop(0, _TCH, zb_zero, 0)

    def elu_inplace(av, bv, n_edges):
        @plsc.parallel_loop(0, n_edges, unroll=16)
        def _(i):
            sl = pl.ds(0, 16)
            v = av[i, sl] + bv[i, sl]
            av[i, sl] = jnp.where(v > 0, v, jnp.exp(v) - 1.0)

    for p in range(_NPASS):
        qa = p
        qb = 4 + p

        def g_start(bi, st):
            r2, c2, gi, hi, av, bv, sem = st
            base = g_lo + bi * _K
            pltpu.sync_copy(row2.at[pl.ds(base, _K)], r2)
            pltpu.sync_copy(col2.at[pl.ds(base, _K)], c2)
            for j in range(_K):
                for u in range(8):
                    sl = pl.ds(u * 16, 16)
                    gi[j, sl] = r2[j, sl] * 8 + qa
                    hi[j, sl] = c2[j, sl] * 8 + qb
            for j in range(_K):
                pltpu.async_copy(tab.at[gi.at[j]],
                                 av.at[pl.ds(j * 128, 128)], sem)
                pltpu.async_copy(tab.at[hi.at[j]],
                                 bv.at[pl.ds(j * 128, 128)], sem)

        def g_finish_compute(st):
            r2, c2, gi, hi, av, bv, sem = st
            for j in range(_K):
                pltpu.make_async_copy(tab.at[gi.at[j]],
                                      av.at[pl.ds(j * 128, 128)], sem).wait()
                pltpu.make_async_copy(tab.at[hi.at[j]],
                                      bv.at[pl.ds(j * 128, 128)], sem).wait()
            elu_inplace(av, bv, _BE)
            for j in range(_K):
                pltpu.sync_copy(av.at[pl.ds(j * 128, 128)],
                                agg.at[r2.at[j]], add=True)

        for j in range(_TROWS // _TCH):
            pltpu.async_copy(zb, agg.at[pl.ds(s * _TROWS + j * _TCH, _TCH)],
                             sga)
        for j in range(_TROWS // _TCH):
            pltpu.make_async_copy(zb,
                                  agg.at[pl.ds(s * _TROWS + j * _TCH, _TCH)],
                                  sga).wait()
        plsc.subcore_barrier()

        @pl.when(nb > 0)
        def _():
            g_start(0, sets[0])

        def pair(k, carry):
            bi0 = 2 * k

            @pl.when(bi0 + 1 < nb)
            def _():
                g_start(bi0 + 1, sets[1])
            g_finish_compute(sets[0])

            @pl.when(bi0 + 2 < nb)
            def _():
                g_start(bi0 + 2, sets[0])

            @pl.when(bi0 + 1 < nb)
            def _():
                g_finish_compute(sets[1])
            return carry
        lax.fori_loop(0, (nb + 1) // 2, pair, 0)

        def tail(g, carry):
            r2, c2, gi, hi, av, bv, sem = sets[0]
            pltpu.sync_copy(row2.at[pl.ds(g, 1)], r2.at[pl.ds(0, 1)])
            pltpu.sync_copy(col2.at[pl.ds(g, 1)], c2.at[pl.ds(0, 1)])
            for u in range(8):
                sl = pl.ds(u * 16, 16)
                gi[0, sl] = r2[0, sl] * 8 + qa
                hi[0, sl] = c2[0, sl] * 8 + qb
            pltpu.sync_copy(tab.at[gi.at[0]], av.at[pl.ds(0, 128)])
            pltpu.sync_copy(tab.at[hi.at[0]], bv.at[pl.ds(0, 128)])
            elu_inplace(av, bv, 128)
            pltpu.sync_copy(av.at[pl.ds(0, 128)], agg.at[r2.at[0]], add=True)
            return carry
        lax.fori_loop(g_lo + nb * _K, g_hi, tail, 0)
        plsc.subcore_barrier()
        off = s * _TROWS
        pltpu.sync_copy(agg.at[pl.ds(off, _TROWS)],
                        out.at[pl.ds(off, _TROWS),
                               pl.ds(c * 64 + p * 16, 16)])
        plsc.subcore_barrier()


def _counts_body(row2, out0, out1, cnt, r16, ones_v, zb1):
    c = lax.axis_index("c")
    s = lax.axis_index("s")
    w = s * 2 + c
    g_lo = (w * _G) // _NW
    g_hi = ((w + 1) * _G) // _NW
    nb = (g_hi - g_lo) // 16

    for j in range(8):
        ones_v[pl.ds(j * 16, 16)] = jnp.full((16,), 1.0, _f32)

    def z(i, carry):
        zb1[pl.ds(i * 16, 16)] = jnp.zeros((16,), _f32)
        return carry
    lax.fori_loop(0, _CROWS // 16, z, 0)

    pltpu.sync_copy(zb1, cnt.at[pl.ds(s * _CROWS, _CROWS)])
    plsc.subcore_barrier()

    def block(b, carry):
        pltpu.sync_copy(row2.at[pl.ds(g_lo + b * 16, 16)], r16)
        for j in range(16):
            pltpu.sync_copy(ones_v, cnt.at[r16.at[j]], add=True)
        return carry
    lax.fori_loop(0, nb, block, 0)

    def tail(g, carry):
        pltpu.sync_copy(row2.at[pl.ds(g, 1)], r16.at[pl.ds(0, 1)])
        pltpu.sync_copy(ones_v, cnt.at[r16.at[0]], add=True)
        return carry
    lax.fori_loop(g_lo + nb * 16, g_hi, tail, 0)
    plsc.subcore_barrier()

    @pl.when(c == 0)
    def _():
        pltpu.sync_copy(cnt.at[pl.ds(s * _CROWS, _CROWS)],
                        out0.at[pl.ds(s * _CROWS, _CROWS)])

    @pl.when(c == 1)
    def _():
        pltpu.sync_copy(cnt.at[pl.ds(s * _CROWS, _CROWS)],
                        out1.at[pl.ds(s * _CROWS, _CROWS)])


_sc_mesh = plsc.VectorSubcoreMesh(core_axis_name="c", subcore_axis_name="s")

_edge_call = pl.kernel(
    _edge_body,
    out_type=jax.ShapeDtypeStruct((_NP, 128), _f32),
    mesh=_sc_mesh,
    compiler_params=pltpu.CompilerParams(use_tc_tiling_on_sc=False),
    scratch_types=[
        pltpu.VMEM_SHARED((_NP, _PW), _f32),
        pltpu.VMEM((_TCH, _PW), _f32),
        pltpu.VMEM((_K, 128), jnp.int32),
        pltpu.VMEM((_K, 128), jnp.int32),
        pltpu.VMEM((_K, 128), jnp.int32),
        pltpu.VMEM((_K, 128), jnp.int32),
        pltpu.VMEM((_BE, _PW), _f32),
        pltpu.VMEM((_BE, _PW), _f32),
        pltpu.VMEM((_K, 128), jnp.int32),
        pltpu.VMEM((_K, 128), jnp.int32),
        pltpu.VMEM((_K, 128), jnp.int32),
        pltpu.VMEM((_K, 128), jnp.int32),
        pltpu.VMEM((_BE, _PW), _f32),
        pltpu.VMEM((_BE, _PW), _f32),
        pltpu.SemaphoreType.DMA,
        pltpu.SemaphoreType.DMA,
    ],
)

_counts_call = pl.kernel(
    _counts_body,
    out_type=[jax.ShapeDtypeStruct((_NPC,), _f32),
              jax.ShapeDtypeStruct((_NPC,), _f32)],
    mesh=_sc_mesh,
    compiler_params=pltpu.CompilerParams(use_tc_tiling_on_sc=False),
    scratch_types=[
        pltpu.VMEM_SHARED((_NPC,), _f32),
        pltpu.VMEM((16, 128), jnp.int32),
        pltpu.VMEM((128,), _f32),
        pltpu.VMEM((_CROWS,), _f32),
    ],
)


def _node0_body(x_ref, w1, b1r, w2, b2r, ab, cb, h_out, hab_out):
    xb = x_ref[...]
    h1 = _elu_tc(jnp.dot(xb, w1[...], preferred_element_type=_f32, precision=lax.Precision.HIGHEST) + b1r[...])
    h = _elu_tc(jnp.dot(h1, w2[...], preferred_element_type=_f32, precision=lax.Precision.HIGHEST) + b2r[...])
    h_out[...] = h
    hab_out[...] = jnp.dot(h, ab[...], preferred_element_type=_f32, precision=lax.Precision.HIGHEST) + cb[...]


def _prep_body(cnt_ref, cf_out):
    cz = cnt_ref[...]
    craw = cz[0] + cz[1]
    r = 1.0 / jnp.maximum(craw, 1.0)
    m = jnp.where(craw > 0, 1.0, 0.0)
    cf_out[...] = jnp.concatenate(
        [jnp.broadcast_to(r, (_TBLK, 64)), jnp.broadcast_to(m, (_TBLK, 64))],
        axis=-1)


def _combine(p_ref, cf_ref, g_ref, be_ref, h_ref):
    P = p_ref[...]
    cf = cf_ref[...]
    psum = P[:, 0:64] + P[:, 64:128]
    return g_ref[...] * (psum * cf[:, 0:64]) + be_ref[...] * cf[:, 64:128] \
        + h_ref[...]


def _mid_body(p_ref, cf_ref, g_ref, be_ref, h_ref, ab, cb, h_out, hab_out):
    h = _combine(p_ref, cf_ref, g_ref, be_ref, h_ref)
    h_out[...] = h
    hab_out[...] = jnp.dot(h, ab[...], preferred_element_type=_f32, precision=lax.Precision.HIGHEST) + cb[...]


def _head_body(p_ref, cf_ref, g_ref, be_ref, h_ref,
               ws, bsr, wq1, bq1r, wq2, bq2r,
               wh1, bh1r, ghs, bhr, wh2, bh2r, wh3, bh3r, out_o):
    h = _combine(p_ref, cf_ref, g_ref, be_ref, h_ref)
    feats = _elu_tc(jnp.dot(h, ws[...], preferred_element_type=_f32, precision=lax.Precision.HIGHEST) + bsr[...])
    cq = _elu_tc(jnp.dot(feats, wq1[...], preferred_element_type=_f32, precision=lax.Precision.HIGHEST) + bq1r[...])
    con = jnp.dot(cq, wq2[...], preferred_element_type=_f32, precision=lax.Precision.HIGHEST) + bq2r[...]
    sv = _elu_tc(jnp.dot(feats, wh1[...], preferred_element_type=_f32, precision=lax.Precision.HIGHEST) + bh1r[...])
    sv = sv * ghs[...] + bhr[...]
    sv = _elu_tc(jnp.dot(sv, wh2[...], preferred_element_type=_f32, precision=lax.Precision.HIGHEST) + bh2r[...])
    lg = jnp.dot(sv, wh3[...], preferred_element_type=_f32, precision=lax.Precision.HIGHEST) + bh3r[...]
    out_o[...] = jnp.concatenate(
        [con, lg, jnp.zeros((_TBLK, 119), _f32)], axis=-1)


def _row_spec(nc):
    return pl.BlockSpec((_TBLK, nc), lambda i: (i, 0))


def _w_spec(shape):
    nd = len(shape)
    return pl.BlockSpec(shape, lambda i: (0,) * nd)


_P_SPEC = pl.BlockSpec((_TBLK, 128), lambda i: (i, 0))
_CNT_SPEC = pl.BlockSpec((2, _TBLK, 1), lambda i: (0, i, 0))
_CF_SPEC = pl.BlockSpec((_TBLK, 128), lambda i: (i, 0))

_prep_call = pl.pallas_call(
    _prep_body,
    grid=(_TGRID,),
    in_specs=[_CNT_SPEC],
    out_specs=[pl.BlockSpec((_TBLK, 128), lambda i: (i, 0))],
    out_shape=[jax.ShapeDtypeStruct((_N, 128), _f32)],
)

_node0_call = pl.pallas_call(
    _node0_body,
    grid=(_TGRID,),
    in_specs=[_row_spec(16), _w_spec((16, 64)), _w_spec((1, 64)),
              _w_spec((64, 64)), _w_spec((1, 64)),
              _w_spec((64, 128)), _w_spec((1, 128))],
    out_specs=[_row_spec(64), _row_spec(128)],
    out_shape=[jax.ShapeDtypeStruct((_N, 64), _f32),
               jax.ShapeDtypeStruct((_N, 128), _f32)],
)

_mid_call = pl.pallas_call(
    _mid_body,
    grid=(_TGRID,),
    in_specs=[_P_SPEC, _CF_SPEC, _w_spec((1, 64)), _w_spec((1, 64)),
              _row_spec(64), _w_spec((64, 128)), _w_spec((1, 128))],
    out_specs=[_row_spec(64), _row_spec(128)],
    out_shape=[jax.ShapeDtypeStruct((_N, 64), _f32),
               jax.ShapeDtypeStruct((_N, 128), _f32)],
)

_head_call = pl.pallas_call(
    _head_body,
    grid=(_TGRID,),
    in_specs=[_P_SPEC, _CF_SPEC, _w_spec((1, 64)), _w_spec((1, 64)),
              _row_spec(64),
              _w_spec((64, 64)), _w_spec((1, 64)),
              _w_spec((64, 32)), _w_spec((1, 32)),
              _w_spec((32, 8)), _w_spec((1, 8)),
              _w_spec((64, 64)), _w_spec((1, 64)),
              _w_spec((1, 64)), _w_spec((1, 64)),
              _w_spec((64, 32)), _w_spec((1, 32)),
              _w_spec((32, 1)), _w_spec((1, 1))],
    out_specs=[_row_spec(128)],
    out_shape=[jax.ShapeDtypeStruct((_N, 128), _f32)],
)


def kernel(x, edge_index, batch, W1, b1, W2, b2, Wc, bc, gamma, beta,
           Ws, bs, Wq1, bq1, Wq2, bq2, Wh1, bh1, gh, bh, Wh2, bh2, Wh3, bh3):
    row2 = edge_index[0].astype(jnp.int32).reshape(_G, 128)
    col2 = edge_index[1].astype(jnp.int32).reshape(_G, 128)

    cnt0, cnt1 = _counts_call(row2)
    cnt3 = jnp.stack([cnt0, cnt1]).reshape(2, _NPC, 1)
    (cf,) = _prep_call(cnt3)

    A = Wc[:, :_H, :] - Wc[:, _H:, :]
    B = Wc[:, _H:, :]
    AB = jnp.concatenate([A, B], axis=2)
    CB = jnp.concatenate([bc, jnp.zeros_like(bc)], axis=1).reshape(4, 1, 128)
    G = (_BN * gamma).reshape(4, 1, 64)
    BE = beta.reshape(4, 1, 64)

    h, hab = _node0_call(
        x, W1, b1.reshape(1, 64), W2, b2.reshape(1, 64), AB[0], CB[0])

    for i in range(4):
        P = _edge_call(hab.reshape(8 * _N, 16), row2, col2)
        if i < 3:
            h, hab = _mid_call(
                P, cf, G[i], BE[i], h, AB[i + 1], CB[i + 1])
        else:
            (hout,) = _head_call(
                P, cf, G[i], BE[i], h,
                Ws, bs.reshape(1, 64),
                Wq1, bq1.reshape(1, 32), Wq2, bq2.reshape(1, 8),
                Wh1, bh1.reshape(1, 64),
                (_BN * gh).reshape(1, 64), bh.reshape(1, 64),
                Wh2, bh2.reshape(1, 32), Wh3, bh3.reshape(1, 1))
    return (hout[:, 0:8], hout[:, 8:9], batch)


# speedup vs baseline: 9.1009x; 1.0685x over previous
import functools

import jax
import jax.numpy as jnp
import numpy as np
from jax import lax
from jax.experimental import pallas as pl
from jax.experimental.pallas import tpu as pltpu
from jax.experimental.pallas import tpu_sc as plsc

_N = 50000
_E = 800000
_H = 64
_BN = 1.0 / np.sqrt(1.0 + 1e-5)
_G = _E // 128
_NW = 32
_TBLK = 2000
_TGRID = _N // _TBLK
_NP = 50048
_TROWS = _NP // 16
_TCH = 136
_NPC = 51200
_CROWS = _NPC // 16

_f32 = jnp.float32


def _elu_tc(v):
    return jnp.where(v > 0, v, jnp.exp(v) - 1.0)


_K = 8
_BE = _K * 128
_PW = 16
_NPASS = 64 // _PW


def _edge_body(tab, row2, col2, out, agg, zb,
               r2a, c2a, ga, ha, ava, bva,
               r2b, c2b, gb, hb, avb, bvb, sga, sgb):
    c = lax.axis_index("c")
    s = lax.axis_index("s")
    w = s * 2 + c
    g_lo = (w * _G) // _NW
    g_hi = ((w + 1) * _G) // _NW
    nb = (g_hi - g_lo) // _K
    sets = ((r2a, c2a, ga, ha, ava, bva, sga),
            (r2b, c2b, gb, hb, avb, bvb, sgb))

    def zb_zero(i, carry):
        zb[i, pl.ds(0, 16)] = jnp.zeros((16,), _f32)
        return carry
    lax.fori_loop(0, _TCH, zb_zero, 0)

    def elu_inplace(av, bv, n_edges):
        @plsc.parallel_loop(0, n_edges, unroll=16)
        def _(i):
            sl = pl.ds(0, 16)
            v = av[i, sl] + bv[i, sl]
            av[i, sl] = jnp.where(v > 0, v, jnp.exp(v) - 1.0)

    for p in range(_NPASS):
        qa = p
        qb = 4 + p

        def g_start(bi, st):
            r2, c2, gi, hi, av, bv, sem = st
            base = g_lo + bi * _K
            pltpu.sync_copy(row2.at[pl.ds(base, _K)], r2)
            pltpu.sync_copy(col2.at[pl.ds(base, _K)], c2)
            for j in range(_K):
                for u in range(8):
                    sl = pl.ds(u * 16, 16)
                    gi[j, sl] = r2[j, sl] * 8 + qa
                    hi[j, sl] = c2[j, sl] * 8 + qb
            for j in range(_K):
                pltpu.async_copy(tab.at[gi.at[j]],
                                 av.at[pl.ds(j * 128, 128)], sem)
                pltpu.async_copy(tab.at[hi.at[j]],
                                 bv.at[pl.ds(j * 128, 128)], sem)

        def g_finish_compute(st):
            r2, c2, gi, hi, av, bv, sem = st
            for j in range(_K):
                pltpu.make_async_copy(tab.at[gi.at[j]],
                                      av.at[pl.ds(j * 128, 128)], sem).wait()
                pltpu.make_async_copy(tab.at[hi.at[j]],
                                      bv.at[pl.ds(j * 128, 128)], sem).wait()
            elu_inplace(av, bv, _BE)
            for j in range(_K):
                pltpu.sync_copy(av.at[pl.ds(j * 128, 128)],
                                agg.at[r2.at[j]], add=True)

        for j in range(_TROWS // _TCH):
            pltpu.async_copy(zb, agg.at[pl.ds(s * _TROWS + j * _TCH, _TCH)],
                             sga)
        for j in range(_TROWS // _TCH):
            pltpu.make_async_copy(zb,
                                  agg.at[pl.ds(s * _TROWS + j * _TCH, _TCH)],
                                  sga).wait()
        plsc.subcore_barrier()

        @pl.when(nb > 0)
        def _():
            g_start(0, sets[0])

        def pair(k, carry):
            bi0 = 2 * k

            @pl.when(bi0 + 1 < nb)
            def _():
                g_start(bi0 + 1, sets[1])
            g_finish_compute(sets[0])

            @pl.when(bi0 + 2 < nb)
            def _():
                g_start(bi0 + 2, sets[0])

            @pl.when(bi0 + 1 < nb)
            def _():
                g_finish_compute(sets[1])
            return carry
        lax.fori_loop(0, (nb + 1) // 2, pair, 0)

        def tail(g, carry):
            r2, c2, gi, hi, av, bv, sem = sets[0]
            pltpu.sync_copy(row2.at[pl.ds(g, 1)], r2.at[pl.ds(0, 1)])
            pltpu.sync_copy(col2.at[pl.ds(g, 1)], c2.at[pl.ds(0, 1)])
            for u in range(8):
                sl = pl.ds(u * 16, 16)
                gi[0, sl] = r2[0, sl] * 8 + qa
                hi[0, sl] = c2[0, sl] * 8 + qb
            pltpu.sync_copy(tab.at[gi.at[0]], av.at[pl.ds(0, 128)])
            pltpu.sync_copy(tab.at[hi.at[0]], bv.at[pl.ds(0, 128)])
            elu_inplace(av, bv, 128)
            pltpu.sync_copy(av.at[pl.ds(0, 128)], agg.at[r2.at[0]], add=True)
            return carry
        lax.fori_loop(g_lo + nb * _K, g_hi, tail, 0)
        plsc.subcore_barrier()
        off = s * _TROWS
        pltpu.sync_copy(agg.at[pl.ds(off, _TROWS)],
                        out.at[pl.ds(off, _TROWS),
                               pl.ds(c * 64 + p * 16, 16)])
        plsc.subcore_barrier()


def _counts_body(row2, out0, out1, cnt, r16, ones_v, zb1):
    c = lax.axis_index("c")
    s = lax.axis_index("s")
    w = s * 2 + c
    g_lo = (w * _G) // _NW
    g_hi = ((w + 1) * _G) // _NW
    nb = (g_hi - g_lo) // 16

    for j in range(8):
        ones_v[pl.ds(j * 16, 16)] = jnp.full((16,), 1.0, _f32)

    def z(i, carry):
        zb1[pl.ds(i * 16, 16)] = jnp.zeros((16,), _f32)
        return carry
    lax.fori_loop(0, _CROWS // 16, z, 0)

    pltpu.sync_copy(zb1, cnt.at[pl.ds(s * _CROWS, _CROWS)])
    plsc.subcore_barrier()

    def block(b, carry):
        pltpu.sync_copy(row2.at[pl.ds(g_lo + b * 16, 16)], r16)
        for j in range(16):
            pltpu.sync_copy(ones_v, cnt.at[r16.at[j]], add=True)
        return carry
    lax.fori_loop(0, nb, block, 0)

    def tail(g, carry):
        pltpu.sync_copy(row2.at[pl.ds(g, 1)], r16.at[pl.ds(0, 1)])
        pltpu.sync_copy(ones_v, cnt.at[r16.at[0]], add=True)
        return carry
    lax.fori_loop(g_lo + nb * 16, g_hi, tail, 0)
    plsc.subcore_barrier()

    @pl.when(c == 0)
    def _():
        pltpu.sync_copy(cnt.at[pl.ds(s * _CROWS, _CROWS)],
                        out0.at[pl.ds(s * _CROWS, _CROWS)])

    @pl.when(c == 1)
    def _():
        pltpu.sync_copy(cnt.at[pl.ds(s * _CROWS, _CROWS)],
                        out1.at[pl.ds(s * _CROWS, _CROWS)])


_sc_mesh = plsc.VectorSubcoreMesh(core_axis_name="c", subcore_axis_name="s")

_edge_call = pl.kernel(
    _edge_body,
    out_type=jax.ShapeDtypeStruct((_NP, 128), _f32),
    mesh=_sc_mesh,
    compiler_params=pltpu.CompilerParams(use_tc_tiling_on_sc=False),
    scratch_types=[
        pltpu.VMEM_SHARED((_NP, _PW), _f32),
        pltpu.VMEM((_TCH, _PW), _f32),
        pltpu.VMEM((_K, 128), jnp.int32),
        pltpu.VMEM((_K, 128), jnp.int32),
        pltpu.VMEM((_K, 128), jnp.int32),
        pltpu.VMEM((_K, 128), jnp.int32),
        pltpu.VMEM((_BE, _PW), _f32),
        pltpu.VMEM((_BE, _PW), _f32),
        pltpu.VMEM((_K, 128), jnp.int32),
        pltpu.VMEM((_K, 128), jnp.int32),
        pltpu.VMEM((_K, 128), jnp.int32),
        pltpu.VMEM((_K, 128), jnp.int32),
        pltpu.VMEM((_BE, _PW), _f32),
        pltpu.VMEM((_BE, _PW), _f32),
        pltpu.SemaphoreType.DMA,
        pltpu.SemaphoreType.DMA,
    ],
)

_counts_call = pl.kernel(
    _counts_body,
    out_type=[jax.ShapeDtypeStruct((_NPC,), _f32),
              jax.ShapeDtypeStruct((_NPC,), _f32)],
    mesh=_sc_mesh,
    compiler_params=pltpu.CompilerParams(use_tc_tiling_on_sc=False),
    scratch_types=[
        pltpu.VMEM_SHARED((_NPC,), _f32),
        pltpu.VMEM((16, 128), jnp.int32),
        pltpu.VMEM((128,), _f32),
        pltpu.VMEM((_CROWS,), _f32),
    ],
)


def _node0_body(x_ref, w1, b1r, w2, b2r, ab, cb, h_out, hab_out):
    hp = lax.Precision.HIGHEST
    xb = x_ref[...]
    h1 = _elu_tc(jnp.dot(xb, w1[...], preferred_element_type=_f32,
                         precision=hp) + b1r[...])
    h = _elu_tc(jnp.dot(h1, w2[...], preferred_element_type=_f32,
                        precision=hp) + b2r[...])
    h_out[...] = h
    hab_out[...] = jnp.dot(h, ab[...], preferred_element_type=_f32,
                           precision=hp) + cb[...]


def _prep_body(cnt_ref, cf_out):
    cz = cnt_ref[...]
    craw = cz[0] + cz[1]
    r = 1.0 / jnp.maximum(craw, 1.0)
    m = jnp.where(craw > 0, 1.0, 0.0)
    cf_out[...] = jnp.concatenate(
        [jnp.broadcast_to(r, (_TBLK, 64)), jnp.broadcast_to(m, (_TBLK, 64))],
        axis=-1)


def _combine(p_ref, cf_ref, g_ref, be_ref, h_ref):
    P = p_ref[...]
    cf = cf_ref[...]
    psum = P[:, 0:64] + P[:, 64:128]
    return g_ref[...] * (psum * cf[:, 0:64]) + be_ref[...] * cf[:, 64:128] \
        + h_ref[...]


def _mid_body(p_ref, cf_ref, g_ref, be_ref, h_ref, ab, cb, h_out, hab_out):
    h = _combine(p_ref, cf_ref, g_ref, be_ref, h_ref)
    h_out[...] = h
    hab_out[...] = jnp.dot(h, ab[...], preferred_element_type=_f32,
                           precision=lax.Precision.HIGHEST) + cb[...]


def _head_body(p_ref, cf_ref, g_ref, be_ref, h_ref,
               ws, bsr, wq1, bq1r, wq2, bq2r,
               wh1, bh1r, ghs, bhr, wh2, bh2r, wh3, bh3r, out_o):
    h = _combine(p_ref, cf_ref, g_ref, be_ref, h_ref)
    feats = _elu_tc(jnp.dot(h, ws[...], preferred_element_type=_f32) + bsr[...])
    cq = _elu_tc(jnp.dot(feats, wq1[...], preferred_element_type=_f32) + bq1r[...])
    con = jnp.dot(cq, wq2[...], preferred_element_type=_f32) + bq2r[...]
    sv = _elu_tc(jnp.dot(feats, wh1[...], preferred_element_type=_f32) + bh1r[...])
    sv = sv * ghs[...] + bhr[...]
    sv = _elu_tc(jnp.dot(sv, wh2[...], preferred_element_type=_f32) + bh2r[...])
    lg = jnp.dot(sv, wh3[...], preferred_element_type=_f32) + bh3r[...]
    out_o[...] = jnp.concatenate(
        [con, lg, jnp.zeros((_TBLK, 119), _f32)], axis=-1)


def _row_spec(nc):
    return pl.BlockSpec((_TBLK, nc), lambda i: (i, 0))


def _w_spec(shape):
    nd = len(shape)
    return pl.BlockSpec(shape, lambda i: (0,) * nd)


_P_SPEC = pl.BlockSpec((_TBLK, 128), lambda i: (i, 0))
_CNT_SPEC = pl.BlockSpec((2, _TBLK, 1), lambda i: (0, i, 0))
_CF_SPEC = pl.BlockSpec((_TBLK, 128), lambda i: (i, 0))

_prep_call = pl.pallas_call(
    _prep_body,
    grid=(_TGRID,),
    in_specs=[_CNT_SPEC],
    out_specs=[pl.BlockSpec((_TBLK, 128), lambda i: (i, 0))],
    out_shape=[jax.ShapeDtypeStruct((_N, 128), _f32)],
)

_node0_call = pl.pallas_call(
    _node0_body,
    grid=(_TGRID,),
    in_specs=[_row_spec(16), _w_spec((16, 64)), _w_spec((1, 64)),
              _w_spec((64, 64)), _w_spec((1, 64)),
              _w_spec((64, 128)), _w_spec((1, 128))],
    out_specs=[_row_spec(64), _row_spec(128)],
    out_shape=[jax.ShapeDtypeStruct((_N, 64), _f32),
               jax.ShapeDtypeStruct((_N, 128), _f32)],
)

_mid_call = pl.pallas_call(
    _mid_body,
    grid=(_TGRID,),
    in_specs=[_P_SPEC, _CF_SPEC, _w_spec((1, 64)), _w_spec((1, 64)),
              _row_spec(64), _w_spec((64, 128)), _w_spec((1, 128))],
    out_specs=[_row_spec(64), _row_spec(128)],
    out_shape=[jax.ShapeDtypeStruct((_N, 64), _f32),
               jax.ShapeDtypeStruct((_N, 128), _f32)],
)

_head_call = pl.pallas_call(
    _head_body,
    grid=(_TGRID,),
    in_specs=[_P_SPEC, _CF_SPEC, _w_spec((1, 64)), _w_spec((1, 64)),
              _row_spec(64),
              _w_spec((64, 64)), _w_spec((1, 64)),
              _w_spec((64, 32)), _w_spec((1, 32)),
              _w_spec((32, 8)), _w_spec((1, 8)),
              _w_spec((64, 64)), _w_spec((1, 64)),
              _w_spec((1, 64)), _w_spec((1, 64)),
              _w_spec((64, 32)), _w_spec((1, 32)),
              _w_spec((32, 1)), _w_spec((1, 1))],
    out_specs=[_row_spec(128)],
    out_shape=[jax.ShapeDtypeStruct((_N, 128), _f32)],
)


def kernel(x, edge_index, batch, W1, b1, W2, b2, Wc, bc, gamma, beta,
           Ws, bs, Wq1, bq1, Wq2, bq2, Wh1, bh1, gh, bh, Wh2, bh2, Wh3, bh3):
    row2 = edge_index[0].astype(jnp.int32).reshape(_G, 128)
    col2 = edge_index[1].astype(jnp.int32).reshape(_G, 128)

    cnt0, cnt1 = _counts_call(row2)
    cnt3 = jnp.stack([cnt0, cnt1]).reshape(2, _NPC, 1)
    (cf,) = _prep_call(cnt3)

    A = Wc[:, :_H, :] - Wc[:, _H:, :]
    B = Wc[:, _H:, :]
    AB = jnp.concatenate([A, B], axis=2)
    CB = jnp.concatenate([bc, jnp.zeros_like(bc)], axis=1).reshape(4, 1, 128)
    G = (_BN * gamma).reshape(4, 1, 64)
    BE = beta.reshape(4, 1, 64)

    h, hab = _node0_call(
        x, W1, b1.reshape(1, 64), W2, b2.reshape(1, 64), AB[0], CB[0])

    for i in range(4):
        P = _edge_call(hab.reshape(8 * _N, 16), row2, col2)
        if i < 3:
            h, hab = _mid_call(
                P, cf, G[i], BE[i], h, AB[i + 1], CB[i + 1])
        else:
            (hout,) = _head_call(
                P, cf, G[i], BE[i], h,
                Ws, bs.reshape(1, 64),
                Wq1, bq1.reshape(1, 32), Wq2, bq2.reshape(1, 8),
                Wh1, bh1.reshape(1, 64),
                (_BN * gh).reshape(1, 64), bh.reshape(1, 64),
                Wh2, bh2.reshape(1, 32), Wh3, bh3.reshape(1, 1))
    return (hout[:, 0:8], hout[:, 8:9], batch)
